# baseline (speedup 1.0000x reference)
import functools

import jax
import jax.numpy as jnp
from jax import lax
from jax.experimental import pallas as pl
from jax.experimental.pallas import tpu as pltpu
from jax.experimental.pallas import tpu_sc as plsc

TOPK = 5
IGN = -100
V = 100000
N = 1024
L = 16
NW = 32
ROWS_PER_W = N // NW
VREGS = V // L
UNROLL = 10


def _body(flat_hbm, targ_hbm, out_hbm, targv, rowbuf, resv):
    wid = lax.axis_index("s") * 2 + lax.axis_index("c")
    base = wid * ROWS_PER_W
    pltpu.sync_copy(targ_hbm.at[pl.ds(base, ROWS_PER_W)], targv)

    lane = lax.iota(jnp.int32, L)
    zero16 = jnp.zeros((L,), jnp.int32)

    def row_body(j, carry):
        cc, vc = carry
        row = base + j
        pltpu.sync_copy(flat_hbm.at[pl.ds(row * V, V)], rowbuf)
        j16 = jnp.full((L,), j, jnp.int32)
        tidx16 = plsc.load_gather(targv, [j16])
        tclamp16 = jnp.maximum(tidx16, 0)
        tval16 = plsc.load_gather(rowbuf, [tclamp16])

        def cnt_body(i, c):
            acc, pos = c
            off = i * (UNROLL * L)
            for u in range(UNROLL):
                x = rowbuf[pl.ds(off + u * L, L)]
                pu = pos + (u * L)
                m = (x > tval16) | ((x == tval16) & (pu < tidx16))
                acc = acc + jnp.where(m, 1, 0).astype(jnp.int32)
            return acc, pos + UNROLL * L

        acc, _ = lax.fori_loop(0, VREGS // UNROLL, cnt_body, (zero16, lane))
        rank = jnp.sum(acc)
        valid = jnp.max(tidx16) != IGN
        hit = (rank < TOPK) & valid
        cc = cc + jnp.where(hit, 1.0, 0.0)
        vc = vc + jnp.where(valid, 1.0, 0.0)
        return cc, vc

    cc, vc = lax.fori_loop(0, ROWS_PER_W, row_body, (jnp.float32(0.0), jnp.float32(0.0)))
    resv[...] = jnp.where(lane == 0, cc, jnp.where(lane == 1, vc, 0.0))
    pltpu.sync_copy(resv, out_hbm.at[wid])


@jax.jit
def kernel(logits, targets):
    flat = logits.reshape(-1)
    tflat = targets.reshape(-1).astype(jnp.int32)
    mesh = plsc.VectorSubcoreMesh(core_axis_name="c", subcore_axis_name="s")
    out = pl.kernel(
        _body,
        out_type=jax.ShapeDtypeStruct((NW, L), jnp.float32),
        mesh=mesh,
        scratch_types=[
            pltpu.VMEM((ROWS_PER_W,), jnp.int32),
            pltpu.VMEM((V,), jnp.float32),
            pltpu.VMEM((L,), jnp.float32),
        ],
        compiler_params=pltpu.CompilerParams(needs_layout_passes=False),
    )(flat, tflat)
    correct = out[:, 0].sum()
    valid = out[:, 1].sum()
    acc = correct / jnp.maximum(valid, 1.0)
    return jnp.where(valid == 0, jnp.float32(0.0), acc).astype(jnp.float32)


# speedup vs baseline: 2.2291x; 2.2291x over previous
import jax
import jax.numpy as jnp
from jax import lax
from jax.experimental import pallas as pl
from jax.experimental.pallas import tpu as pltpu
from jax.experimental.pallas import tpu_sc as plsc

TOPK = 5
IGN = -100
V = 100000
N = 1024
L = 16
NW = 32
ROWS_PER_W = N // NW
C = V // 2
CV = C // L
UNROLL = 8


def _body(flat_hbm, targ_hbm, out_hbm, targv, idxv, tvals, buf0, buf1, resv,
          sem_g, sem_c):
    wid = lax.axis_index("s") * 2 + lax.axis_index("c")
    base = wid * ROWS_PER_W

    pltpu.async_copy(flat_hbm.at[pl.ds(base * V, C)], buf0, sem_c)

    pltpu.sync_copy(targ_hbm.at[pl.ds(base, ROWS_PER_W)], targv)
    lane = lax.iota(jnp.int32, L)
    for jj in range(ROWS_PER_W // L):
        t16 = targv[pl.ds(jj * L, L)]
        rows16 = jnp.full((L,), base + jj * L, jnp.int32) + lane
        idxv[pl.ds(jj * L, L)] = rows16 * V + jnp.maximum(t16, 0)
    pltpu.async_copy(flat_hbm.at[idxv], tvals, sem_g).wait()

    bufs = (buf0, buf1)
    zero16 = jnp.zeros((L,), jnp.int32)

    def row_body(j, carry):
        cc, vc = carry
        row = base + j
        j16 = jnp.full((L,), j, jnp.int32)
        tidx16 = plsc.load_gather(targv, [j16])
        tval16 = plsc.load_gather(tvals, [j16])
        t = jnp.max(tidx16)
        tc = jnp.clip(t, 0, V - 1)

        acc = zero16
        for c in range(2):
            buf = bufs[c]
            o = c * C
            pltpu.make_async_copy(flat_hbm.at[pl.ds(0, C)], buf, sem_c).wait()
            nxt = jnp.where(c == 0, row * V + C, (row + 1) * V)
            nxt = jnp.minimum(nxt, N * V - C)
            pltpu.async_copy(flat_hbm.at[pl.ds(nxt, C)], bufs[1 - c], sem_c)

            s = jnp.clip(tc - o, 0, C)
            fs = s // L

            @plsc.parallel_loop(0, fs, unroll=UNROLL, carry=acc)
            def ge_loop(i, a):
                x = buf[pl.ds(i * L, L)]
                return a + jnp.where(x >= tval16, 1, 0).astype(jnp.int32)
            acc = ge_loop

            fm = jnp.minimum(fs, CV - 1)
            x = buf[pl.ds(fm * L, L)]
            posv = jnp.full((L,), o + fm * L, jnp.int32) + lane
            m = (x > tval16) | ((x == tval16) & (posv < tidx16))
            m = m & (jnp.full((L,), fs, jnp.int32) < CV)
            acc = acc + jnp.where(m, 1, 0).astype(jnp.int32)

            @plsc.parallel_loop(fs + 1, CV, unroll=UNROLL, carry=acc)
            def gt_loop(i, a):
                x = buf[pl.ds(i * L, L)]
                return a + jnp.where(x > tval16, 1, 0).astype(jnp.int32)
            acc = gt_loop

        rank = jnp.sum(acc)
        valid = t != IGN
        hit = (rank < TOPK) & valid
        cc = cc + jnp.where(hit, 1.0, 0.0)
        vc = vc + jnp.where(valid, 1.0, 0.0)
        return cc, vc

    cc, vc = lax.fori_loop(0, ROWS_PER_W, row_body,
                           (jnp.float32(0.0), jnp.float32(0.0)))
    pltpu.make_async_copy(flat_hbm.at[pl.ds(0, C)], buf0, sem_c).wait()

    resv[...] = jnp.where(lane == 0, cc, jnp.where(lane == 1, vc, 0.0))
    pltpu.sync_copy(resv, out_hbm.at[wid])


@jax.jit
def kernel(logits, targets):
    flat = logits.reshape(-1)
    tflat = targets.reshape(-1).astype(jnp.int32)
    mesh = plsc.VectorSubcoreMesh(core_axis_name="c", subcore_axis_name="s")
    out = pl.kernel(
        _body,
        out_type=jax.ShapeDtypeStruct((NW, L), jnp.float32),
        mesh=mesh,
        scratch_types=[
            pltpu.VMEM((ROWS_PER_W,), jnp.int32),
            pltpu.VMEM((ROWS_PER_W,), jnp.int32),
            pltpu.VMEM((ROWS_PER_W,), jnp.float32),
            pltpu.VMEM((C,), jnp.float32),
            pltpu.VMEM((C,), jnp.float32),
            pltpu.VMEM((L,), jnp.float32),
            pltpu.SemaphoreType.DMA,
            pltpu.SemaphoreType.DMA,
        ],
        compiler_params=pltpu.CompilerParams(needs_layout_passes=False),
    )(flat, tflat)
    correct = out[:, 0].sum()
    valid = out[:, 1].sum()
    acc = correct / jnp.maximum(valid, 1.0)
    return jnp.where(valid == 0, jnp.float32(0.0), acc).astype(jnp.float32)
